# Initial kernel scaffold; baseline (speedup 1.0000x reference)
import jax
import jax.numpy as jnp
from jax.experimental import pallas as pl


def kernel(x_src, x_dst, edge_label_index):
    raise NotImplementedError("write your pallas kernel here")


# speedup vs baseline: 1.2013x; 1.2013x over previous
import functools

import jax
import jax.numpy as jnp
from jax import lax
from jax.experimental import pallas as pl
from jax.experimental.pallas import tpu as pltpu
from jax.experimental.pallas import tpu_sc as plsc

D = 128
NB = 320000
NC = 2
NS = 16
NW = NC * NS
BPW = NB // NW
E = 400
NCHUNK = BPW // E


def _sc_body(xsrc_hbm, xdst_hbm, row_hbm, col_hbm, out_hbm,
             ridx_v, cidx_v, src_v, dst_v, out_v, sem_s, sem_d):
    wid = lax.axis_index("s") * NC + lax.axis_index("c")
    lanes = lax.iota(jnp.int32, 16)

    def chunk_body(c, carry):
        base = wid * BPW + c * E
        pltpu.sync_copy(row_hbm.at[pl.ds(base, E)], ridx_v)
        pltpu.sync_copy(col_hbm.at[pl.ds(base, E)], cidx_v)
        cp_s = pltpu.async_copy(xsrc_hbm.at[ridx_v], src_v, sem_s)
        cp_d = pltpu.async_copy(xdst_hbm.at[cidx_v], dst_v, sem_d)
        cp_s.wait()
        cp_d.wait()

        def group_body(g, carry2):
            eids = g * 16 + lanes

            def d_body(j, acc):
                for u in range(8):
                    dv = jnp.full((16,), j * 8 + u, jnp.int32)
                    a = plsc.load_gather(src_v, [eids, dv])
                    b = plsc.load_gather(dst_v, [eids, dv])
                    acc = acc + a * b
                return acc

            acc = lax.fori_loop(0, D // 8, d_body, jnp.zeros((16,), jnp.float32))
            out_v[pl.ds(g * 16, 16)] = acc
            return carry2

        lax.fori_loop(0, E // 16, group_body, 0)
        pltpu.sync_copy(out_v, out_hbm.at[pl.ds(base, E)])
        return carry

    lax.fori_loop(0, NCHUNK, chunk_body, 0)


@jax.jit
def _scores(x_src, x_dst, row, col):
    mesh = plsc.VectorSubcoreMesh(core_axis_name="c", subcore_axis_name="s")
    return pl.kernel(
        _sc_body,
        out_type=jax.ShapeDtypeStruct((NB,), jnp.float32),
        mesh=mesh,
        scratch_types=[
            pltpu.VMEM((E,), jnp.int32),
            pltpu.VMEM((E,), jnp.int32),
            pltpu.VMEM((E, D), jnp.float32),
            pltpu.VMEM((E, D), jnp.float32),
            pltpu.VMEM((E,), jnp.float32),
            pltpu.SemaphoreType.DMA,
            pltpu.SemaphoreType.DMA,
        ],
        compiler_params=pltpu.CompilerParams(needs_layout_passes=False),
    )(x_src, x_dst, row, col)


def kernel(x_src, x_dst, edge_label_index):
    row = edge_label_index[0].astype(jnp.int32)
    col = edge_label_index[1].astype(jnp.int32)
    return _scores(x_src, x_dst, row, col)


# speedup vs baseline: 6.6045x; 5.4976x over previous
import jax
import jax.numpy as jnp
from jax import lax
from jax.experimental import pallas as pl
from jax.experimental.pallas import tpu as pltpu
from jax.experimental.pallas import tpu_sc as plsc

D = 128
NB = 320000
NC = 2
NS = 16
NW = NC * NS
BPW = NB // NW
E = 200
NCHUNK = BPW // E


def _sc_body(xsrc_hbm, xdst_hbm, row_hbm, col_hbm, out_hbm,
             ridx, cidx, sv0, sv1, dv0, dv1, ov0, ov1,
             ss0, ss1, sd0, sd1, so0, so1):
    sv = (sv0, sv1)
    dv = (dv0, dv1)
    ov = (ov0, ov1)
    ss = (ss0, ss1)
    sd = (sd0, sd1)
    so = (so0, so1)
    wid = lax.axis_index("s") * NC + lax.axis_index("c")
    base = wid * BPW
    lanes = lax.iota(jnp.int32, 16)
    last_lane = lanes == 15
    pltpu.sync_copy(row_hbm.at[pl.ds(base, BPW)], ridx)
    pltpu.sync_copy(col_hbm.at[pl.ds(base, BPW)], cidx)
    pltpu.async_copy(xsrc_hbm.at[ridx.at[pl.ds(0, E)]], sv[0], ss[0])
    pltpu.async_copy(xdst_hbm.at[cidx.at[pl.ds(0, E)]], dv[0], sd[0])

    @pl.loop(0, NCHUNK, step=2)
    def pair(c0):
        for b in range(2):
            c = c0 + b
            nxt = c + 1

            @pl.when(nxt < NCHUNK)
            def _():
                off = nxt * E
                pltpu.async_copy(
                    xsrc_hbm.at[ridx.at[pl.ds(off, E)]], sv[1 - b], ss[1 - b])
                pltpu.async_copy(
                    xdst_hbm.at[cidx.at[pl.ds(off, E)]], dv[1 - b], sd[1 - b])

            pltpu.make_async_copy(
                xsrc_hbm.at[ridx.at[pl.ds(0, E)]], sv[b], ss[b]).wait()
            pltpu.make_async_copy(
                xdst_hbm.at[cidx.at[pl.ds(0, E)]], dv[b], sd[b]).wait()

            @pl.when(c >= 2)
            def _():
                pltpu.make_async_copy(
                    ov[b], out_hbm.at[pl.ds(0, E)], so[b]).wait()

            @pl.loop(0, E, unroll=2)
            def edge(e):
                acc0 = jnp.zeros((16,), jnp.float32)
                acc1 = jnp.zeros((16,), jnp.float32)
                for u in range(0, 8, 2):
                    acc0 = acc0 + (sv[b][e, pl.ds(16 * u, 16)]
                                   * dv[b][e, pl.ds(16 * u, 16)])
                    acc1 = acc1 + (sv[b][e, pl.ds(16 * (u + 1), 16)]
                                   * dv[b][e, pl.ds(16 * (u + 1), 16)])
                cum = jnp.cumsum(acc0 + acc1)
                evec = jnp.full((16,), e, jnp.int32)
                plsc.store_scatter(ov[b], [evec], cum, mask=last_lane)

            pltpu.async_copy(ov[b], out_hbm.at[pl.ds(base + c * E, E)], so[b])

    pltpu.make_async_copy(ov[0], out_hbm.at[pl.ds(0, E)], so[0]).wait()
    pltpu.make_async_copy(ov[1], out_hbm.at[pl.ds(0, E)], so[1]).wait()


@jax.jit
def _scores(x_src, x_dst, row, col):
    mesh = plsc.VectorSubcoreMesh(core_axis_name="c", subcore_axis_name="s")
    return pl.kernel(
        _sc_body,
        out_type=jax.ShapeDtypeStruct((NB,), jnp.float32),
        mesh=mesh,
        scratch_types=[
            pltpu.VMEM((BPW,), jnp.int32),
            pltpu.VMEM((BPW,), jnp.int32),
            pltpu.VMEM((E, D), jnp.float32),
            pltpu.VMEM((E, D), jnp.float32),
            pltpu.VMEM((E, D), jnp.float32),
            pltpu.VMEM((E, D), jnp.float32),
            pltpu.VMEM((E,), jnp.float32),
            pltpu.VMEM((E,), jnp.float32),
            pltpu.SemaphoreType.DMA,
            pltpu.SemaphoreType.DMA,
            pltpu.SemaphoreType.DMA,
            pltpu.SemaphoreType.DMA,
            pltpu.SemaphoreType.DMA,
            pltpu.SemaphoreType.DMA,
        ],
        compiler_params=pltpu.CompilerParams(needs_layout_passes=False),
    )(x_src, x_dst, row, col)


def kernel(x_src, x_dst, edge_label_index):
    row = edge_label_index[0].astype(jnp.int32)
    col = edge_label_index[1].astype(jnp.int32)
    return _scores(x_src, x_dst, row, col)
